# Initial kernel scaffold; baseline (speedup 1.0000x reference)
import functools

import jax
import jax.numpy as jnp
from jax import lax
from jax.experimental import pallas as pl
from jax.experimental.pallas import tpu as pltpu
from jax.experimental.pallas import tpu_sc as plsc

N_NODES = 10000
D = 128
LANES = 16
N_TILES = 32
EDGE_BLK = 128
NP = 10016
ROWS_PER_TILE = NP // 16


def _deg_body(n_chunks, dst_hbm, out_hbm, idx_v, acc_v):
    c = lax.axis_index("c")
    s = lax.axis_index("s")
    wid = c * 16 + s
    pltpu.sync_copy(dst_hbm.at[wid], idx_v)

    def zero(i, carry):
        acc_v[pl.ds(i * LANES, LANES)] = jnp.zeros((LANES,), jnp.float32)
        return carry

    lax.fori_loop(0, NP // LANES, zero, 0)
    ones = jnp.ones((LANES,), jnp.float32)

    def body(j, carry):
        for k in range(EDGE_BLK // LANES):
            d16 = idx_v[j, pl.ds(k * LANES, LANES)]
            plsc.addupdate_scatter(acc_v, [d16], ones)
        return carry

    lax.fori_loop(0, n_chunks, body, 0)
    pltpu.sync_copy(acc_v, out_hbm.at[wid])


def _prop1_body(n_chunks, g_hbm, src_hbm, dst_hbm, out_hbm, src_v, dst_v,
                g_v, acc_v):
    c = lax.axis_index("c")
    s = lax.axis_index("s")
    wid = c * 16 + s
    pltpu.sync_copy(src_hbm.at[wid], src_v)
    pltpu.sync_copy(dst_hbm.at[wid], dst_v)
    pltpu.sync_copy(g_hbm, g_v)

    def zero(i, carry):
        acc_v[pl.ds(i * LANES, LANES)] = jnp.zeros((LANES,), jnp.float32)
        return carry

    lax.fori_loop(0, NP // LANES, zero, 0)

    def body(j, carry):
        for k in range(EDGE_BLK // LANES):
            s16 = src_v[j, pl.ds(k * LANES, LANES)]
            d16 = dst_v[j, pl.ds(k * LANES, LANES)]
            vals = plsc.load_gather(g_v, [s16])
            plsc.addupdate_scatter(acc_v, [d16], vals)
        return carry

    lax.fori_loop(0, n_chunks, body, 0)
    pltpu.sync_copy(acc_v, out_hbm.at[wid])


def _prop_wide_body(n_chunks, g_hbm, src_hbm, dst_hbm, out_hbm, src_v, dst_v,
                    buf_v, acc_sh, sem):
    c = lax.axis_index("c")
    s = lax.axis_index("s")
    wid = c * 16 + s
    pltpu.sync_copy(src_hbm.at[wid], src_v)
    pltpu.sync_copy(dst_hbm.at[wid], dst_v)

    def zero(i, carry):
        for k in range(D // LANES):
            buf_v[i, pl.ds(k * LANES, LANES)] = jnp.zeros((LANES,), jnp.float32)
        return carry

    lax.fori_loop(0, EDGE_BLK, zero, 0)
    base = s * ROWS_PER_TILE
    for i in range(ROWS_PER_TILE // EDGE_BLK):
        pltpu.sync_copy(buf_v, acc_sh.at[pl.ds(base + i * EDGE_BLK, EDGE_BLK)])
    rem = ROWS_PER_TILE % EDGE_BLK
    if rem:
        pltpu.sync_copy(buf_v.at[pl.ds(0, rem)],
                        acc_sh.at[pl.ds(base + ROWS_PER_TILE - rem, rem)])
    plsc.subcore_barrier()

    def body(j, carry):
        pltpu.async_copy(g_hbm.at[src_v.at[j]], buf_v, sem).wait()
        pltpu.sync_copy(buf_v, acc_sh.at[dst_v.at[j]], add=True)
        return carry

    lax.fori_loop(0, n_chunks, body, 0)
    plsc.subcore_barrier()
    pltpu.sync_copy(acc_sh.at[pl.ds(base, ROWS_PER_TILE)],
                    out_hbm.at[c].at[pl.ds(base, ROWS_PER_TILE)])


def _tc1_body(x_ref, w_ref, degp_ref, g_ref, dinv_ref):
    deg = jnp.sum(degp_ref[...], axis=0) + 1.0
    dinv = lax.rsqrt(deg)
    h = jnp.dot(x_ref[...], w_ref[...], preferred_element_type=jnp.float32)
    g_ref[...] = h * dinv[:, None]
    dinv_ref[...] = dinv[:, None]


def _tc_mid_body(p_ref, g_ref, dinv_ref, b_ref, w_ref, gout_ref):
    dinv = dinv_ref[...]
    out = (p_ref[0] + p_ref[1] + g_ref[...]) * dinv + b_ref[...]
    h = jnp.dot(jnp.maximum(out, 0.0), w_ref[...],
                preferred_element_type=jnp.float32)
    gout_ref[...] = h * dinv


def _tc4_body(s3p_ref, g3_ref, dinv_ref, b3_ref, out_ref):
    agg = jnp.sum(s3p_ref[...], axis=0)[:, None]
    z = (agg + g3_ref[...]) * dinv_ref[...] + b3_ref[...]
    out_ref[...] = jax.nn.sigmoid(z)


def _tc_call(body, out_shapes, *args):
    return pl.pallas_call(
        body,
        out_shape=out_shapes,
    )(*args)


def kernel(x, edge_index, W1, b1, W2, b2, W3, b3):
    src = edge_index[0].astype(jnp.int32)
    dst = edge_index[1].astype(jnp.int32)
    n_edges = src.shape[0]
    per_tile = -(-n_edges // (N_TILES * EDGE_BLK)) * EDGE_BLK
    n_chunks = per_tile // EDGE_BLK
    e_pad = per_tile * N_TILES
    src_p = jnp.concatenate(
        [src, jnp.zeros((e_pad - n_edges,), jnp.int32)]
    ).reshape(N_TILES, n_chunks, EDGE_BLK)
    dst_p = jnp.concatenate(
        [dst, jnp.full((e_pad - n_edges,), N_NODES, jnp.int32)]
    ).reshape(N_TILES, n_chunks, EDGE_BLK)
    x_p = jnp.pad(x, ((0, NP - N_NODES), (0, 0)))

    mesh = plsc.VectorSubcoreMesh(core_axis_name="c", subcore_axis_name="s")

    deg_parts = pl.kernel(
        functools.partial(_deg_body, n_chunks),
        out_type=jax.ShapeDtypeStruct((N_TILES, NP), jnp.float32),
        mesh=mesh,
        scratch_types=[
            pltpu.VMEM((n_chunks, EDGE_BLK), jnp.int32),
            pltpu.VMEM((NP,), jnp.float32),
        ],
    )(dst_p)

    prop_wide = pl.kernel(
        functools.partial(_prop_wide_body, n_chunks),
        out_type=jax.ShapeDtypeStruct((2, NP, D), jnp.float32),
        mesh=mesh,
        scratch_types=[
            pltpu.VMEM((n_chunks, EDGE_BLK), jnp.int32),
            pltpu.VMEM((n_chunks, EDGE_BLK), jnp.int32),
            pltpu.VMEM((EDGE_BLK, D), jnp.float32),
            pltpu.VMEM_SHARED((NP, D), jnp.float32),
            pltpu.SemaphoreType.DMA,
        ],
    )

    prop1 = pl.kernel(
        functools.partial(_prop1_body, n_chunks),
        out_type=jax.ShapeDtypeStruct((N_TILES, NP), jnp.float32),
        mesh=mesh,
        scratch_types=[
            pltpu.VMEM((n_chunks, EDGE_BLK), jnp.int32),
            pltpu.VMEM((n_chunks, EDGE_BLK), jnp.int32),
            pltpu.VMEM((NP,), jnp.float32),
            pltpu.VMEM((NP,), jnp.float32),
        ],
    )

    g1, dinv = _tc_call(
        _tc1_body,
        (jax.ShapeDtypeStruct((NP, D), jnp.float32),
         jax.ShapeDtypeStruct((NP, 1), jnp.float32)),
        x_p, W1, deg_parts)

    p1 = prop_wide(g1, src_p, dst_p)
    g2 = _tc_call(
        _tc_mid_body,
        jax.ShapeDtypeStruct((NP, D), jnp.float32),
        p1, g1, dinv, b1.reshape(1, D), W2)

    p2 = prop_wide(g2, src_p, dst_p)
    g3 = _tc_call(
        _tc_mid_body,
        jax.ShapeDtypeStruct((NP, 1), jnp.float32),
        p2, g2, dinv, b2.reshape(1, D), W3)

    s3_parts = prop1(g3.reshape(NP), src_p, dst_p)
    out = _tc_call(
        _tc4_body,
        jax.ShapeDtypeStruct((NP, 1), jnp.float32),
        s3_parts, g3, dinv, b3.reshape(1, 1))
    return out[:N_NODES]


# speedup vs baseline: 17.5944x; 17.5944x over previous
import functools

import jax
import jax.numpy as jnp
from jax import lax
from jax.experimental import pallas as pl
from jax.experimental.pallas import tpu as pltpu
from jax.experimental.pallas import tpu_sc as plsc

N_NODES = 10000
D = 128
LANES = 16
N_TILES = 32
EDGE_BLK = 128
NP = 10112
ROWS_PER_TILE = NP // 16


def _deg_body(n_chunks, dst_hbm, out_hbm, idx_v, acc_v):
    c = lax.axis_index("c")
    s = lax.axis_index("s")
    wid = c * 16 + s
    pltpu.sync_copy(dst_hbm.at[wid], idx_v)

    def zero(i, carry):
        acc_v[0, pl.ds(i * LANES, LANES)] = jnp.zeros((LANES,), jnp.float32)
        return carry

    lax.fori_loop(0, NP // LANES, zero, 0)
    ones = jnp.ones((LANES,), jnp.float32)
    z16 = jnp.zeros((LANES,), jnp.int32)

    def body(j, carry):
        for k in range(EDGE_BLK // LANES):
            d16 = idx_v[j, pl.ds(k * LANES, LANES)]
            plsc.addupdate_scatter(acc_v, [z16, d16], ones)
        return carry

    lax.fori_loop(0, n_chunks, body, 0)
    pltpu.sync_copy(acc_v, out_hbm.at[wid])


def _prop1_body(n_chunks, g_hbm, src_hbm, dst_hbm, out_hbm, src_v, dst_v,
                g_v, acc_v):
    c = lax.axis_index("c")
    s = lax.axis_index("s")
    wid = c * 16 + s
    pltpu.sync_copy(src_hbm.at[wid], src_v)
    pltpu.sync_copy(dst_hbm.at[wid], dst_v)
    pltpu.sync_copy(g_hbm, g_v)

    def zero(i, carry):
        acc_v[0, pl.ds(i * LANES, LANES)] = jnp.zeros((LANES,), jnp.float32)
        return carry

    lax.fori_loop(0, NP // LANES, zero, 0)
    z16 = jnp.zeros((LANES,), jnp.int32)

    def body(j, carry):
        for k in range(EDGE_BLK // LANES):
            s16 = src_v[j, pl.ds(k * LANES, LANES)]
            d16 = dst_v[j, pl.ds(k * LANES, LANES)]
            vals = plsc.load_gather(g_v, [s16])
            plsc.addupdate_scatter(acc_v, [z16, d16], vals)
        return carry

    lax.fori_loop(0, n_chunks, body, 0)
    pltpu.sync_copy(acc_v, out_hbm.at[wid])


def _prop_wide_body(n_chunks, g_hbm, src_hbm, dst_hbm, out_hbm, src_v, dst_v,
                    buf_v, acc_sh, sem):
    c = lax.axis_index("c")
    s = lax.axis_index("s")
    wid = c * 16 + s
    pltpu.sync_copy(src_hbm.at[wid], src_v)
    pltpu.sync_copy(dst_hbm.at[wid], dst_v)

    def zero(i, carry):
        for k in range(D // LANES):
            buf_v[i, pl.ds(k * LANES, LANES)] = jnp.zeros((LANES,), jnp.float32)
        return carry

    lax.fori_loop(0, EDGE_BLK, zero, 0)
    base = s * ROWS_PER_TILE
    for i in range(ROWS_PER_TILE // EDGE_BLK):
        pltpu.sync_copy(buf_v, acc_sh.at[pl.ds(base + i * EDGE_BLK, EDGE_BLK)])
    rem = ROWS_PER_TILE % EDGE_BLK
    if rem:
        pltpu.sync_copy(buf_v.at[pl.ds(0, rem)],
                        acc_sh.at[pl.ds(base + ROWS_PER_TILE - rem, rem)])
    plsc.subcore_barrier()

    def body(j, carry):
        pltpu.async_copy(g_hbm.at[src_v.at[j]], buf_v, sem).wait()
        pltpu.sync_copy(buf_v, acc_sh.at[dst_v.at[j]], add=True)
        return carry

    lax.fori_loop(0, n_chunks, body, 0)
    plsc.subcore_barrier()
    pltpu.sync_copy(acc_sh.at[pl.ds(base, ROWS_PER_TILE)],
                    out_hbm.at[c].at[pl.ds(base, ROWS_PER_TILE)])


def _tc1_body(x_ref, w_ref, degp_ref, g_ref, dinv_ref):
    deg = jnp.sum(degp_ref[...], axis=(0, 1)) + 1.0
    dinv = lax.rsqrt(deg)
    h = jnp.dot(x_ref[...], w_ref[...], preferred_element_type=jnp.float32)
    g_ref[...] = h * dinv[:, None]
    dinv_ref[...] = dinv[:, None]


def _tc_mid_body(p_ref, g_ref, dinv_ref, b_ref, w_ref, gout_ref):
    dinv = dinv_ref[...]
    out = (p_ref[0] + p_ref[1] + g_ref[...]) * dinv + b_ref[...]
    h = jnp.dot(jnp.maximum(out, 0.0), w_ref[...],
                preferred_element_type=jnp.float32)
    gout_ref[...] = h * dinv


def _tc4_body(s3p_ref, g3_ref, dinv_ref, b3_ref, out_ref):
    agg = jnp.sum(s3p_ref[...], axis=(0, 1))[:, None]
    z = (agg + g3_ref[...]) * dinv_ref[...] + b3_ref[...]
    out_ref[...] = jax.nn.sigmoid(z)


def _tc_call(body, out_shapes, *args):
    return pl.pallas_call(
        body,
        out_shape=out_shapes,
    )(*args)


def kernel(x, edge_index, W1, b1, W2, b2, W3, b3):
    src = edge_index[0].astype(jnp.int32)
    dst = edge_index[1].astype(jnp.int32)
    n_edges = src.shape[0]
    per_tile = -(-n_edges // (N_TILES * EDGE_BLK)) * EDGE_BLK
    n_chunks = per_tile // EDGE_BLK
    e_pad = per_tile * N_TILES
    src_p = jnp.concatenate(
        [src, jnp.zeros((e_pad - n_edges,), jnp.int32)]
    ).reshape(N_TILES, n_chunks, EDGE_BLK)
    dst_p = jnp.concatenate(
        [dst, jnp.full((e_pad - n_edges,), N_NODES, jnp.int32)]
    ).reshape(N_TILES, n_chunks, EDGE_BLK)
    x_p = jnp.pad(x, ((0, NP - N_NODES), (0, 0)))

    mesh = plsc.VectorSubcoreMesh(core_axis_name="c", subcore_axis_name="s")
    sc_params = pltpu.CompilerParams(needs_layout_passes=False)

    deg_parts = pl.kernel(
        functools.partial(_deg_body, n_chunks),
        out_type=jax.ShapeDtypeStruct((N_TILES, 1, NP), jnp.float32),
        mesh=mesh,
        scratch_types=[
            pltpu.VMEM((n_chunks, EDGE_BLK), jnp.int32),
            pltpu.VMEM((1, NP), jnp.float32),
        ],
        compiler_params=sc_params,
    )(dst_p)

    prop_wide = pl.kernel(
        functools.partial(_prop_wide_body, n_chunks),
        out_type=jax.ShapeDtypeStruct((2, NP, D), jnp.float32),
        mesh=mesh,
        scratch_types=[
            pltpu.VMEM((n_chunks, EDGE_BLK), jnp.int32),
            pltpu.VMEM((n_chunks, EDGE_BLK), jnp.int32),
            pltpu.VMEM((EDGE_BLK, D), jnp.float32),
            pltpu.VMEM_SHARED((NP, D), jnp.float32),
            pltpu.SemaphoreType.DMA,
        ],
        compiler_params=sc_params,
    )

    prop1 = pl.kernel(
        functools.partial(_prop1_body, n_chunks),
        out_type=jax.ShapeDtypeStruct((N_TILES, 1, NP), jnp.float32),
        mesh=mesh,
        scratch_types=[
            pltpu.VMEM((n_chunks, EDGE_BLK), jnp.int32),
            pltpu.VMEM((n_chunks, EDGE_BLK), jnp.int32),
            pltpu.VMEM((NP,), jnp.float32),
            pltpu.VMEM((1, NP), jnp.float32),
        ],
        compiler_params=sc_params,
    )

    g1, dinv = _tc_call(
        _tc1_body,
        (jax.ShapeDtypeStruct((NP, D), jnp.float32),
         jax.ShapeDtypeStruct((NP, 1), jnp.float32)),
        x_p, W1, deg_parts)

    p1 = prop_wide(g1, src_p, dst_p)
    g2 = _tc_call(
        _tc_mid_body,
        jax.ShapeDtypeStruct((NP, D), jnp.float32),
        p1, g1, dinv, b1.reshape(1, D), W2)

    p2 = prop_wide(g2, src_p, dst_p)
    g3 = _tc_call(
        _tc_mid_body,
        jax.ShapeDtypeStruct((NP, 1), jnp.float32),
        p2, g2, dinv, b2.reshape(1, D), W3)

    s3_parts = prop1(g3.reshape(NP), src_p, dst_p)
    out = _tc_call(
        _tc4_body,
        jax.ShapeDtypeStruct((NP, 1), jnp.float32),
        s3_parts, g3, dinv, b3.reshape(1, 1))
    return out[:N_NODES]
